# baseline (device time: 170570 ns/iter reference)
import jax
import jax.numpy as jnp
from jax import lax
from jax.experimental import pallas as pl
from jax.experimental.pallas import tpu as pltpu

N_DEV = 4
M_PER = 1024
N_GLOBAL = 8192
N_PER = N_GLOBAL // N_DEV
K = 4096
SUB = 512
SUBS = N_PER // SUB


def _fused(x, w_mat):
    OFFS = (1, 3, 2)

    def body(
        x_ref,
        w_ref,
        yT_ref,
        gmax_ref,
        xtmp,
        xb,
        wbuf,
        ybuf,
        rbuf,
        arecv,
        amax_tile,
        acc,
        xsems,
        wsems,
        osems,
        ysend_sems,
        yrecv_sems,
        asend_sems,
        arecv_sems,
    ):
        my_i = lax.axis_index("i")

        barrier = pltpu.get_barrier_semaphore()
        for off in (1, 2, 3):
            pl.semaphore_signal(
                barrier,
                inc=1,
                device_id=((my_i + off) % N_DEV,),
                device_id_type=pl.DeviceIdType.MESH,
            )
        pl.semaphore_wait(barrier, 3)

        dests = [(my_i + off) % N_DEV for off in OFFS] + [my_i]

        def w_copy(s, buf):
            dst = dests[s // SUBS]
            col = dst * N_PER + (s % SUBS) * SUB
            return pltpu.make_async_copy(
                w_ref.at[:, pl.ds(col, SUB)],
                wbuf.at[buf],
                wsems.at[buf],
            )

        wcopies = {0: w_copy(0, 0), 1: w_copy(1, 1)}
        wcopies[0].start()
        wcopies[1].start()

        acc[0, 0] = 0.0

        xcopies = []
        for c in range(4):
            cp = pltpu.make_async_copy(
                x_ref.at[:, pl.ds(c * 1024, 1024)],
                xtmp.at[c % 2],
                xsems.at[c % 2],
            )
            if c < 2:
                cp.start()
            xcopies.append(cp)
        for c in range(4):
            xcopies[c].wait()
            xb[:, pl.ds(c * 1024, 1024)] = xtmp[c % 2].astype(jnp.bfloat16)
            if c + 2 < 4:
                xcopies[c + 2].start()

        rdmas = []
        for t in range(N_DEV):
            slot = t % 3
            if t == 3:
                rdmas[0].wait_send()
            for sub in range(SUBS):
                s = t * SUBS + sub
                wcopies[s].wait()
                yblk = jnp.dot(
                    xb[...],
                    wbuf[s % 2].astype(jnp.bfloat16),
                    preferred_element_type=jnp.float32,
                )
                acc[0, 0] = jnp.maximum(acc[0, 0], jnp.max(jnp.abs(yblk)))
                ybuf[slot, :, pl.ds(sub * SUB, SUB)] = yblk.astype(
                    jnp.bfloat16
                )
                if s + 2 < N_DEV * SUBS:
                    nxt = w_copy(s + 2, s % 2)
                    nxt.start()
                    wcopies[s + 2] = nxt
            if t < 3:
                r = pltpu.make_async_remote_copy(
                    src_ref=ybuf.at[slot],
                    dst_ref=rbuf.at[t],
                    send_sem=ysend_sems.at[t],
                    recv_sem=yrecv_sems.at[t],
                    device_id=(dests[t],),
                    device_id_type=pl.DeviceIdType.MESH,
                )
                r.start()
                rdmas.append(r)

        cp_loc = pltpu.make_async_copy(
            ybuf.at[0], yT_ref.at[my_i], osems.at[3]
        )
        cp_loc.start()

        amax_tile[...] = jnp.full((8, 128), acc[0, 0], jnp.float32)
        amax_rdmas = []
        for off in (1, 2, 3):
            r = pltpu.make_async_remote_copy(
                src_ref=amax_tile,
                dst_ref=arecv.at[off - 1],
                send_sem=asend_sems.at[off - 1],
                recv_sem=arecv_sems.at[off - 1],
                device_id=((my_i + off) % N_DEV,),
                device_id_type=pl.DeviceIdType.MESH,
            )
            r.start()
            amax_rdmas.append(r)
        for r in amax_rdmas:
            r.wait_send()
            r.wait_recv()
        gmax_ref[...] = jnp.maximum(
            jnp.max(amax_tile[...]), jnp.max(arecv[...])
        ) * jnp.ones((8, 128), jnp.float32)

        out_stores = [cp_loc]
        for t in range(3):
            recv = pltpu.make_async_remote_copy(
                src_ref=ybuf.at[t],
                dst_ref=rbuf.at[t],
                send_sem=ysend_sems.at[t],
                recv_sem=yrecv_sems.at[t],
                device_id=(my_i,),
                device_id_type=pl.DeviceIdType.MESH,
            )
            recv.wait_recv()
            src_dev = (my_i - OFFS[t]) % N_DEV
            st = pltpu.make_async_copy(
                rbuf.at[t], yT_ref.at[src_dev], osems.at[t]
            )
            st.start()
            out_stores.append(st)
        for st in out_stores:
            st.wait()
        for r in rdmas[1:]:
            r.wait_send()

    return pl.pallas_call(
        body,
        in_specs=[
            pl.BlockSpec(memory_space=pl.ANY),
            pl.BlockSpec(memory_space=pl.ANY),
        ],
        out_specs=[
            pl.BlockSpec(memory_space=pl.ANY),
            pl.BlockSpec(memory_space=pltpu.VMEM),
        ],
        out_shape=[
            jax.ShapeDtypeStruct((N_DEV, M_PER, N_PER), jnp.bfloat16),
            jax.ShapeDtypeStruct((8, 128), jnp.float32),
        ],
        scratch_shapes=[
            pltpu.VMEM((2, M_PER, 1024), jnp.float32),
            pltpu.VMEM((M_PER, K), jnp.bfloat16),
            pltpu.VMEM((2, K, SUB), jnp.float32),
            pltpu.VMEM((3, M_PER, N_PER), jnp.bfloat16),
            pltpu.VMEM((3, M_PER, N_PER), jnp.bfloat16),
            pltpu.VMEM((3, 8, 128), jnp.float32),
            pltpu.VMEM((8, 128), jnp.float32),
            pltpu.SMEM((1, 1), jnp.float32),
            pltpu.SemaphoreType.DMA((2,)),
            pltpu.SemaphoreType.DMA((2,)),
            pltpu.SemaphoreType.DMA((4,)),
            pltpu.SemaphoreType.DMA((3,)),
            pltpu.SemaphoreType.DMA((3,)),
            pltpu.SemaphoreType.DMA((3,)),
            pltpu.SemaphoreType.DMA((3,)),
        ],
        compiler_params=pltpu.CompilerParams(
            collective_id=0, vmem_limit_bytes=64 * 1024 * 1024
        ),
    )(x, w_mat)


def _qdq(yT, gmax):

    def body(y_ref, gmax_ref, out_ref):
        gmax = gmax_ref[0, 0]
        scale = gmax / 127.0
        inv_scale = 127.0 / gmax
        q = jnp.clip(
            jnp.round(y_ref[0].astype(jnp.float32) * inv_scale),
            -127.0,
            127.0,
        )
        out_ref[...] = q * scale

    return pl.pallas_call(
        body,
        grid=(N_DEV,),
        in_specs=[
            pl.BlockSpec((1, M_PER, N_PER), lambda p: (p, 0, 0)),
            pl.BlockSpec((8, 128), lambda p: (0, 0)),
        ],
        out_specs=pl.BlockSpec((M_PER, N_PER), lambda p: (p, 0)),
        out_shape=jax.ShapeDtypeStruct((N_DEV * M_PER, N_PER), jnp.float32),
        compiler_params=pltpu.CompilerParams(
            vmem_limit_bytes=64 * 1024 * 1024
        ),
    )(yT, gmax)


def kernel(x, w_mat):
    yT, gmax = _fused(x, w_mat)
    return _qdq(yT, gmax)


# device time: 162806 ns/iter; 1.0477x vs baseline; 1.0477x over previous
import jax
import jax.numpy as jnp
from jax import lax
from jax.experimental import pallas as pl
from jax.experimental.pallas import tpu as pltpu

N_DEV = 4
M_PER = 1024
N_GLOBAL = 8192
N_PER = N_GLOBAL // N_DEV
K = 4096
NBLK = 512
N_STEPS = N_GLOBAL // NBLK


def _gemm(x, w_mat):

    def body(x_ref, w_ref, y_ref, amax_ref, acc, xb):
        j = pl.program_id(0)

        @pl.when(j == 0)
        def _():
            xb[...] = x_ref[...].astype(jnp.bfloat16)

        yblk = jnp.dot(
            xb[...],
            w_ref[...].astype(jnp.bfloat16),
            preferred_element_type=jnp.float32,
        )
        y_ref[...] = yblk.astype(jnp.bfloat16)
        m = jnp.max(jnp.abs(yblk))

        @pl.when(j == 0)
        def _():
            acc[0, 0] = m

        @pl.when(j > 0)
        def _():
            acc[0, 0] = jnp.maximum(acc[0, 0], m)

        amax_ref[...] = jnp.full((8, 128), acc[0, 0], jnp.float32)

    return pl.pallas_call(
        body,
        grid=(N_STEPS,),
        in_specs=[
            pl.BlockSpec((M_PER, K), lambda j: (0, 0)),
            pl.BlockSpec((K, NBLK), lambda j: (0, j)),
        ],
        out_specs=[
            pl.BlockSpec((M_PER, NBLK), lambda j: (0, j)),
            pl.BlockSpec((8, 128), lambda j: (0, 0)),
        ],
        out_shape=[
            jax.ShapeDtypeStruct((M_PER, N_GLOBAL), jnp.bfloat16),
            jax.ShapeDtypeStruct((8, 128), jnp.float32),
        ],
        scratch_shapes=[
            pltpu.SMEM((1, 1), jnp.float32),
            pltpu.VMEM((M_PER, K), jnp.bfloat16),
        ],
        compiler_params=pltpu.CompilerParams(
            vmem_limit_bytes=64 * 1024 * 1024
        ),
    )(x, w_mat)


def _a2a(y, amax):

    def body(
        y_ref,
        amax_ref,
        qall_ref,
        gmax_ref,
        yv,
        qsend,
        qrecv,
        arecv,
        copy_sems,
        qcopy_sems,
        asend_sems,
        arecv_sems,
        bsend_sems,
        brecv_sems,
    ):
        my_i = lax.axis_index("i")

        barrier = pltpu.get_barrier_semaphore()
        for off in (1, 2, 3):
            pl.semaphore_signal(
                barrier,
                inc=1,
                device_id=((my_i + off) % N_DEV,),
                device_id_type=pl.DeviceIdType.MESH,
            )
        pl.semaphore_wait(barrier, 3)

        amax_rdmas = []
        for off in (1, 2, 3):
            r = pltpu.make_async_remote_copy(
                src_ref=amax_ref,
                dst_ref=arecv.at[off - 1],
                send_sem=asend_sems.at[off - 1],
                recv_sem=arecv_sems.at[off - 1],
                device_id=((my_i + off) % N_DEV,),
                device_id_type=pl.DeviceIdType.MESH,
            )
            r.start()
            amax_rdmas.append(r)

        order = (1, 3, 2)
        stage = {}
        for off in order:
            dst = (my_i + off) % N_DEV
            cp = pltpu.make_async_copy(
                y_ref.at[:, pl.ds(dst * N_PER, N_PER)],
                yv.at[off - 1],
                copy_sems.at[off - 1],
            )
            cp.start()
            stage[off] = cp
        cp_loc = pltpu.make_async_copy(
            y_ref.at[:, pl.ds(my_i * N_PER, N_PER)],
            yv.at[3],
            copy_sems.at[3],
        )
        cp_loc.start()

        for r in amax_rdmas:
            r.wait_send()
            r.wait_recv()
        gmax = jnp.maximum(jnp.max(amax_ref[...]), jnp.max(arecv[...]))
        gmax_ref[...] = jnp.full((8, 128), gmax, jnp.float32)
        inv_scale = 127.0 / gmax

        def quantize(v):
            q = jnp.clip(
                jnp.round(v.astype(jnp.float32) * inv_scale), -127.0, 127.0
            )
            return q.astype(jnp.int8)

        block_rdmas = []
        for off in order:
            dst = (my_i + off) % N_DEV
            stage[off].wait()
            qsend[off - 1] = quantize(yv[off - 1])
            r = pltpu.make_async_remote_copy(
                src_ref=qsend.at[off - 1],
                dst_ref=qrecv.at[off - 1],
                send_sem=bsend_sems.at[off - 1],
                recv_sem=brecv_sems.at[off - 1],
                device_id=(dst,),
                device_id_type=pl.DeviceIdType.MESH,
            )
            r.start()
            block_rdmas.append(r)

        cp_loc.wait()
        qsend[3] = quantize(yv[3])
        cp_q = pltpu.make_async_copy(
            qsend.at[3], qall_ref.at[my_i], qcopy_sems.at[3]
        )
        cp_q.start()

        out_stores = [cp_q]
        for d in (0, 2, 1):
            recv = pltpu.make_async_remote_copy(
                src_ref=qsend.at[d],
                dst_ref=qrecv.at[d],
                send_sem=bsend_sems.at[d],
                recv_sem=brecv_sems.at[d],
                device_id=(my_i,),
                device_id_type=pl.DeviceIdType.MESH,
            )
            recv.wait_recv()
            src_dev = (my_i - d - 1) % N_DEV
            st = pltpu.make_async_copy(
                qrecv.at[d], qall_ref.at[src_dev], qcopy_sems.at[d]
            )
            st.start()
            out_stores.append(st)
        for st in out_stores:
            st.wait()
        for r in block_rdmas:
            r.wait_send()

    return pl.pallas_call(
        body,
        in_specs=[
            pl.BlockSpec(memory_space=pl.ANY),
            pl.BlockSpec(memory_space=pltpu.VMEM),
        ],
        out_specs=[
            pl.BlockSpec(memory_space=pl.ANY),
            pl.BlockSpec(memory_space=pltpu.VMEM),
        ],
        out_shape=[
            jax.ShapeDtypeStruct((N_DEV, M_PER, N_PER), jnp.int8),
            jax.ShapeDtypeStruct((8, 128), jnp.float32),
        ],
        scratch_shapes=[
            pltpu.VMEM((N_DEV, M_PER, N_PER), jnp.bfloat16),
            pltpu.VMEM((N_DEV, M_PER, N_PER), jnp.int8),
            pltpu.VMEM((3, M_PER, N_PER), jnp.int8),
            pltpu.VMEM((3, 8, 128), jnp.float32),
            pltpu.SemaphoreType.DMA((4,)),
            pltpu.SemaphoreType.DMA((4,)),
            pltpu.SemaphoreType.DMA((3,)),
            pltpu.SemaphoreType.DMA((3,)),
            pltpu.SemaphoreType.DMA((3,)),
            pltpu.SemaphoreType.DMA((3,)),
        ],
        compiler_params=pltpu.CompilerParams(
            collective_id=0, vmem_limit_bytes=64 * 1024 * 1024
        ),
    )(y, amax)


def _dequant(qall, gmax):

    def body(q_ref, gmax_ref, out_ref):
        scale = gmax_ref[0, 0] / 127.0
        out_ref[...] = q_ref[0].astype(jnp.float32) * scale

    return pl.pallas_call(
        body,
        grid=(N_DEV,),
        in_specs=[
            pl.BlockSpec((1, M_PER, N_PER), lambda p: (p, 0, 0)),
            pl.BlockSpec((8, 128), lambda p: (0, 0)),
        ],
        out_specs=pl.BlockSpec((M_PER, N_PER), lambda p: (p, 0)),
        out_shape=jax.ShapeDtypeStruct((N_DEV * M_PER, N_PER), jnp.float32),
        compiler_params=pltpu.CompilerParams(
            vmem_limit_bytes=64 * 1024 * 1024
        ),
    )(qall, gmax)


def kernel(x, w_mat):
    y, amax = _gemm(x, w_mat)
    qall, gmax = _a2a(y, amax)
    return _dequant(qall, gmax)
